# baseline (speedup 1.0000x reference)
import functools

import jax
import jax.numpy as jnp
from jax import lax
from jax.experimental import pallas as pl
from jax.experimental.pallas import tpu as pltpu
from jax.experimental.pallas import tpu_sc as plsc

D_M = 64
KNN = 16
R = 256
N = 2048
B = 4
NW = 32


def _dist_kernel(xyzt_ref, q_ref, out_ref):
    q = q_ref[0]
    ptst = xyzt_ref[0]
    d = None
    for c in range(3):
        t = (q[:, c:c + 1] - ptst[c:c + 1, :]) ** 2
        d = t if d is None else d + t
    out_ref[0] = d


def _sc_knn_body(d_hbm, xyzt_hbm, out_hbm, xs_v, drow_v, cand_v, ob_v, sem):
    nc2 = 2
    wid = lax.axis_index("s") * nc2 + lax.axis_index("c")
    b = wid // 8
    blk = wid % 8

    pltpu.async_copy(xyzt_hbm.at[b], xs_v, sem).wait()

    lanes = lax.iota(jnp.int32, 16)
    inf = jnp.float32(jnp.inf)

    def row_body(ri, _):
        row = blk * R + ri
        pltpu.async_copy(d_hbm.at[b, row], drow_v, sem).wait()

        def p1(c, carry):
            mval = carry
            v = drow_v[pl.ds(c * 16, 16)]
            return jnp.minimum(mval, v)
        mval = lax.fori_loop(0, N // 16, p1, jnp.full((16,), inf))
        tau = jnp.broadcast_to(jnp.max(mval), (16,))

        def p2(c, ptr):
            v = drow_v[pl.ds(c * 16, 16)]
            mask = v <= tau
            pos = plsc.cumsum(mask.astype(jnp.int32))
            addr = ptr + pos - 1
            plsc.store_scatter(cand_v, [addr], c * 16 + lanes, mask=mask)
            return ptr + plsc.all_reduce_population_count(mask)
        ptr = lax.fori_loop(0, N // 16, p2, jnp.zeros((16,), jnp.int32))
        ncand = jnp.max(ptr)
        nchunk = (ncand + 15) // 16

        def p3(c, carry):
            top_d, top_i = carry
            mask = (c * 16 + lanes) < ptr
            idxs = cand_v[pl.ds(c * 16, 16)]
            dv = plsc.load_gather(drow_v, [idxs], mask=mask)
            dv = jnp.where(mask, dv, inf)
            sk, sv = plsc.sort_key_val(dv, idxs)
            rk = lax.rev(sk, (0,))
            rv = lax.rev(sv, (0,))
            keep = top_d <= rk
            nd = jnp.where(keep, top_d, rk)
            ni = jnp.where(keep, top_i, rv)
            nk, nv = plsc.sort_key_val(nd, ni)
            return (nk, nv)
        top_d, top_i = lax.fori_loop(
            0, nchunk, p3,
            (jnp.full((16,), inf), jnp.zeros((16,), jnp.int32)))

        rsplat = jnp.broadcast_to(row, (16,))
        base = lanes * (R * 4) + ri * 4
        for c in range(3):
            csplat = jnp.full((16,), c, jnp.int32)
            qc = plsc.load_gather(xs_v, [csplat, rsplat])
            nb = plsc.load_gather(xs_v, [csplat, top_i])
            plsc.store_scatter(ob_v, [base + c], qc - nb)
        plsc.store_scatter(ob_v, [base + 3], jnp.zeros((16,), jnp.float32))
        return 0

    lax.fori_loop(0, R, row_body, 0)
    pltpu.async_copy(ob_v, out_hbm.at[wid], sem).wait()


def _mlp_kernel(delta_ref, x_ref, w1t_ref, b1_ref, w2t_ref, b2_ref, out_ref):
    delta = delta_ref[0].reshape(KNN * R, 4)
    h = jnp.maximum(
        jnp.dot(delta, w1t_ref[...], preferred_element_type=jnp.float32)
        + b1_ref[...], 0.0)
    pe = (jnp.dot(h, w2t_ref[...], preferred_element_type=jnp.float32)
          + b2_ref[...])
    out_ref[0] = x_ref[0] + pe.reshape(KNN, R, D_M)


@jax.jit
def kernel(xyz, x, W1, b1, W2, b2):
    pts = jnp.concatenate(
        [xyz, jnp.zeros((B, N, 5), dtype=xyz.dtype)], axis=-1)
    ptst = jnp.transpose(pts, (0, 2, 1))
    xyzt = ptst[:, :3, :]

    dists = pl.pallas_call(
        _dist_kernel,
        grid=(B, N // R),
        in_specs=[
            pl.BlockSpec((1, 8, N), lambda b_, i: (b_, 0, 0)),
            pl.BlockSpec((1, R, 8), lambda b_, i: (b_, i, 0)),
        ],
        out_specs=pl.BlockSpec((1, R, N), lambda b_, i: (b_, i, 0)),
        out_shape=jax.ShapeDtypeStruct((B, N, N), jnp.float32),
    )(ptst, pts)

    mesh = plsc.VectorSubcoreMesh(core_axis_name="c", subcore_axis_name="s")
    sc_knn = functools.partial(
        pl.kernel,
        mesh=mesh,
        out_type=pltpu.HBM((NW, KNN * R * 4), jnp.float32),
        scratch_types=[
            pltpu.VMEM((3, N), jnp.float32),
            pltpu.VMEM((N,), jnp.float32),
            pltpu.VMEM((N,), jnp.int32),
            pltpu.VMEM((KNN * R * 4,), jnp.float32),
            pltpu.SemaphoreType.DMA,
        ],
        compiler_params=pltpu.CompilerParams(needs_layout_passes=False),
    )(_sc_knn_body)
    delta = sc_knn(dists, xyzt).reshape(NW, KNN, R, 4)

    w1t = jnp.concatenate(
        [W1.T, jnp.zeros((1, D_M), dtype=W1.dtype)], axis=0)
    return pl.pallas_call(
        _mlp_kernel,
        grid=(NW,),
        in_specs=[
            pl.BlockSpec((1, KNN, R, 4), lambda w: (w, 0, 0, 0)),
            pl.BlockSpec((1, KNN, R, D_M), lambda w: (w // 8, 0, w % 8, 0)),
            pl.BlockSpec((4, D_M), lambda w: (0, 0)),
            pl.BlockSpec((1, D_M), lambda w: (0, 0)),
            pl.BlockSpec((D_M, D_M), lambda w: (0, 0)),
            pl.BlockSpec((1, D_M), lambda w: (0, 0)),
        ],
        out_specs=pl.BlockSpec((1, KNN, R, D_M), lambda w: (w // 8, 0, w % 8, 0)),
        out_shape=jax.ShapeDtypeStruct(x.shape, x.dtype),
    )(delta, x, w1t, b1.reshape(1, D_M), W2.T, b2.reshape(1, D_M))


# speedup vs baseline: 1.4047x; 1.4047x over previous
import functools

import jax
import jax.numpy as jnp
from jax import lax
from jax.experimental import pallas as pl
from jax.experimental.pallas import tpu as pltpu
from jax.experimental.pallas import tpu_sc as plsc

D_M = 64
KNN = 16
R = 256
N = 2048
B = 4
NW = 32


def _dist_kernel(xyzt_ref, q_ref, out_ref):
    q = q_ref[0]
    ptst = xyzt_ref[0]
    d = None
    for c in range(3):
        t = (q[:, c:c + 1] - ptst[c:c + 1, :]) ** 2
        d = t if d is None else d + t
    out_ref[0] = d


def _sc_knn_body(d_hbm, xyzt_hbm, out_hbm, xs_v, drow0_v, drow1_v, cand_v,
                 ob_v, sem, sem0, sem1):
    nc2 = 2
    wid = lax.axis_index("s") * nc2 + lax.axis_index("c")
    b = wid // 8
    blk = wid % 8

    pltpu.async_copy(xyzt_hbm.at[b], xs_v, sem).wait()

    lanes = lax.iota(jnp.int32, 16)
    inf = jnp.float32(jnp.inf)

    def process(ri, drow_v):
        row = blk * R + ri

        def p1(c, carry):
            return jnp.minimum(carry, drow_v[pl.ds(c * 16, 16)])
        mval = lax.fori_loop(0, N // 16, p1, jnp.full((16,), inf),
                             unroll=8)
        tau = jnp.broadcast_to(jnp.max(mval), (16,))

        def p2(c, ptr):
            v = drow_v[pl.ds(c * 16, 16)]
            mask = v <= tau
            pos = plsc.cumsum(mask.astype(jnp.int32))
            addr = ptr + pos - 1
            plsc.store_scatter(cand_v, [addr], c * 16 + lanes, mask=mask)
            return ptr + plsc.all_reduce_population_count(mask)
        ptr = lax.fori_loop(0, N // 16, p2, jnp.zeros((16,), jnp.int32),
                            unroll=4)
        ncand = jnp.max(ptr)
        nchunk = (ncand + 15) // 16

        def p3(c, carry):
            top_d, top_i = carry
            mask = (c * 16 + lanes) < ptr
            idxs = cand_v[pl.ds(c * 16, 16)]
            dv = plsc.load_gather(drow_v, [idxs], mask=mask)
            dv = jnp.where(mask, dv, inf)
            sk, sv = plsc.sort_key_val(dv, idxs)
            rk = lax.rev(sk, (0,))
            rv = lax.rev(sv, (0,))
            keep = top_d <= rk
            nd = jnp.where(keep, top_d, rk)
            ni = jnp.where(keep, top_i, rv)
            nk, nv = plsc.sort_key_val(nd, ni)
            return (nk, nv)
        top_d, top_i = lax.fori_loop(
            0, nchunk, p3,
            (jnp.full((16,), inf), jnp.zeros((16,), jnp.int32)))

        rsplat = jnp.broadcast_to(row, (16,))
        base = lanes * (R * 4) + ri * 4
        for c in range(3):
            csplat = jnp.full((16,), c, jnp.int32)
            qc = plsc.load_gather(xs_v, [csplat, rsplat])
            nb = plsc.load_gather(xs_v, [csplat, top_i])
            plsc.store_scatter(ob_v, [base + c], qc - nb)
        plsc.store_scatter(ob_v, [base + 3], jnp.zeros((16,), jnp.float32))

    pltpu.async_copy(d_hbm.at[b, blk * R], drow0_v, sem0)
    pltpu.async_copy(d_hbm.at[b, blk * R + 1], drow1_v, sem1)

    def pair_body(i2, _):
        for phase, (buf, sm) in enumerate(((drow0_v, sem0),
                                           (drow1_v, sem1))):
            ri = i2 * 2 + phase
            pltpu.make_async_copy(d_hbm.at[b, 0], buf, sm).wait()
            process(ri, buf)
            nxt = jnp.minimum(blk * R + ri + 2, N - 1)
            pltpu.async_copy(d_hbm.at[b, nxt], buf, sm)
        return 0

    lax.fori_loop(0, R // 2, pair_body, 0)
    pltpu.make_async_copy(d_hbm.at[b, 0], drow0_v, sem0).wait()
    pltpu.make_async_copy(d_hbm.at[b, 0], drow1_v, sem1).wait()
    pltpu.async_copy(ob_v, out_hbm.at[wid], sem).wait()


def _mlp_kernel(delta_ref, x_ref, w1t_ref, b1_ref, w2t_ref, b2_ref, out_ref):
    delta = delta_ref[0].reshape(KNN * R, 4)
    h = jnp.maximum(
        jnp.dot(delta, w1t_ref[...], preferred_element_type=jnp.float32)
        + b1_ref[...], 0.0)
    pe = (jnp.dot(h, w2t_ref[...], preferred_element_type=jnp.float32)
          + b2_ref[...])
    out_ref[0] = x_ref[0] + pe.reshape(KNN, R, D_M)


@jax.jit
def kernel(xyz, x, W1, b1, W2, b2):
    pts = jnp.concatenate(
        [xyz, jnp.zeros((B, N, 5), dtype=xyz.dtype)], axis=-1)
    ptst = jnp.transpose(pts, (0, 2, 1))
    xyzt = ptst[:, :3, :]

    dists = pl.pallas_call(
        _dist_kernel,
        grid=(B, N // R),
        in_specs=[
            pl.BlockSpec((1, 8, N), lambda b_, i: (b_, 0, 0)),
            pl.BlockSpec((1, R, 8), lambda b_, i: (b_, i, 0)),
        ],
        out_specs=pl.BlockSpec((1, R, N), lambda b_, i: (b_, i, 0)),
        out_shape=jax.ShapeDtypeStruct((B, N, N), jnp.float32),
    )(ptst, pts)

    mesh = plsc.VectorSubcoreMesh(core_axis_name="c", subcore_axis_name="s")
    sc_knn = functools.partial(
        pl.kernel,
        mesh=mesh,
        out_type=pltpu.HBM((NW, KNN * R * 4), jnp.float32),
        scratch_types=[
            pltpu.VMEM((3, N), jnp.float32),
            pltpu.VMEM((N,), jnp.float32),
            pltpu.VMEM((N,), jnp.float32),
            pltpu.VMEM((N,), jnp.int32),
            pltpu.VMEM((KNN * R * 4,), jnp.float32),
            pltpu.SemaphoreType.DMA,
            pltpu.SemaphoreType.DMA,
            pltpu.SemaphoreType.DMA,
        ],
        compiler_params=pltpu.CompilerParams(needs_layout_passes=False),
    )(_sc_knn_body)
    delta = sc_knn(dists, xyzt).reshape(NW, KNN, R, 4)

    w1t = jnp.concatenate(
        [W1.T, jnp.zeros((1, D_M), dtype=W1.dtype)], axis=0)
    return pl.pallas_call(
        _mlp_kernel,
        grid=(NW,),
        in_specs=[
            pl.BlockSpec((1, KNN, R, 4), lambda w: (w, 0, 0, 0)),
            pl.BlockSpec((1, KNN, R, D_M), lambda w: (w // 8, 0, w % 8, 0)),
            pl.BlockSpec((4, D_M), lambda w: (0, 0)),
            pl.BlockSpec((1, D_M), lambda w: (0, 0)),
            pl.BlockSpec((D_M, D_M), lambda w: (0, 0)),
            pl.BlockSpec((1, D_M), lambda w: (0, 0)),
        ],
        out_specs=pl.BlockSpec((1, KNN, R, D_M), lambda w: (w // 8, 0, w % 8, 0)),
        out_shape=jax.ShapeDtypeStruct(x.shape, x.dtype),
    )(delta, x, w1t, b1.reshape(1, D_M), W2.T, b2.reshape(1, D_M))


# speedup vs baseline: 2.3056x; 1.6413x over previous
import functools

import jax
import jax.numpy as jnp
from jax import lax
from jax.experimental import pallas as pl
from jax.experimental.pallas import tpu as pltpu
from jax.experimental.pallas import tpu_sc as plsc

D_M = 64
KNN = 16
N = 2048
R = 256
RS = 128
NW = 32
BSC = 2


def _tc_fused_kernel(xyzt_ref, hilo_ref, q_ref, x_ref, w1t_ref, b1_ref,
                     w2t_ref, b2_ref, out_ref):
    n = xyzt_ref.shape[2]
    ptst = xyzt_ref[0]
    hilo = hilo_ref[0]
    q = q_ref[0]

    d = None
    for c in range(3):
        t = (q[:, c:c + 1] - ptst[c:c + 1, :]) ** 2
        d = t if d is None else d + t

    inf = jnp.float32(jnp.inf)
    d0 = d
    ms = []
    for _ in range(KNN):
        m = jnp.min(d, axis=1, keepdims=True)
        d = jnp.where(d == m, inf, d)
        ms.append(m)

    deltas = []
    for k in range(KNN):
        sel = (d0 == ms[k]).astype(jnp.bfloat16)
        g2 = jnp.dot(sel, hilo, preferred_element_type=jnp.float32)
        deltas.append(q - (g2[:, :8] + g2[:, 8:]))
    delta = jnp.concatenate(deltas, axis=0)

    h = jnp.maximum(
        jnp.dot(delta, w1t_ref[...], preferred_element_type=jnp.float32)
        + b1_ref[...], 0.0)
    pe = (jnp.dot(h, w2t_ref[...], preferred_element_type=jnp.float32)
          + b2_ref[...])
    out_ref[0] = x_ref[0] + pe.reshape(KNN, R, D_M)


def _dist_kernel(xyzt_ref, q_ref, out_ref):
    q = q_ref[0]
    ptst = xyzt_ref[0]
    d = None
    for c in range(3):
        t = (q[:, c:c + 1] - ptst[c:c + 1, :]) ** 2
        d = t if d is None else d + t
    out_ref[0] = d


def _sc_knn_body(d_hbm, xyzt_hbm, out_hbm, xs_v, drow0_v, drow1_v, cand_v,
                 ob_v, sem, sem0, sem1):
    nc2 = 2
    wid = lax.axis_index("s") * nc2 + lax.axis_index("c")
    nblk = N // RS
    b = wid // nblk
    blk = wid % nblk

    pltpu.async_copy(xyzt_hbm.at[b], xs_v, sem).wait()

    lanes = lax.iota(jnp.int32, 16)
    inf = jnp.float32(jnp.inf)

    def process(ri, drow_v):
        row = blk * RS + ri

        def p1(c, carry):
            return jnp.minimum(carry, drow_v[pl.ds(c * 16, 16)])
        mval = lax.fori_loop(0, N // 16, p1, jnp.full((16,), inf),
                             unroll=8)
        tau = jnp.broadcast_to(jnp.max(mval), (16,))

        def p2(c, ptr):
            v = drow_v[pl.ds(c * 16, 16)]
            mask = v <= tau
            pos = plsc.cumsum(mask.astype(jnp.int32))
            addr = ptr + pos - 1
            plsc.store_scatter(cand_v, [addr], c * 16 + lanes, mask=mask)
            return ptr + plsc.all_reduce_population_count(mask)
        ptr = lax.fori_loop(0, N // 16, p2, jnp.zeros((16,), jnp.int32),
                            unroll=4)
        ncand = jnp.max(ptr)
        nchunk = (ncand + 15) // 16

        def p3(c, carry):
            top_d, top_i = carry
            mask = (c * 16 + lanes) < ptr
            idxs = cand_v[pl.ds(c * 16, 16)]
            dv = plsc.load_gather(drow_v, [idxs], mask=mask)
            dv = jnp.where(mask, dv, inf)
            sk, sv = plsc.sort_key_val(dv, idxs)
            rk = lax.rev(sk, (0,))
            rv = lax.rev(sv, (0,))
            keep = top_d <= rk
            nd = jnp.where(keep, top_d, rk)
            ni = jnp.where(keep, top_i, rv)
            nk, nv = plsc.sort_key_val(nd, ni)
            return (nk, nv)
        top_d, top_i = lax.fori_loop(
            0, nchunk, p3,
            (jnp.full((16,), inf), jnp.zeros((16,), jnp.int32)))

        rsplat = jnp.broadcast_to(row, (16,))
        base = lanes * (RS * 4) + ri * 4
        for c in range(3):
            csplat = jnp.full((16,), c, jnp.int32)
            qc = plsc.load_gather(xs_v, [csplat, rsplat])
            nb = plsc.load_gather(xs_v, [csplat, top_i])
            plsc.store_scatter(ob_v, [base + c], qc - nb)
        plsc.store_scatter(ob_v, [base + 3], jnp.zeros((16,), jnp.float32))

    pltpu.async_copy(d_hbm.at[b, blk * RS], drow0_v, sem0)
    pltpu.async_copy(d_hbm.at[b, blk * RS + 1], drow1_v, sem1)

    def pair_body(i2, _):
        for phase, (buf, sm) in enumerate(((drow0_v, sem0),
                                           (drow1_v, sem1))):
            ri = i2 * 2 + phase
            pltpu.make_async_copy(d_hbm.at[b, 0], buf, sm).wait()
            process(ri, buf)
            nxt = jnp.minimum(blk * RS + ri + 2, N - 1)
            pltpu.async_copy(d_hbm.at[b, nxt], buf, sm)
        return 0

    lax.fori_loop(0, RS // 2, pair_body, 0)
    pltpu.make_async_copy(d_hbm.at[b, 0], drow0_v, sem0).wait()
    pltpu.make_async_copy(d_hbm.at[b, 0], drow1_v, sem1).wait()
    pltpu.async_copy(ob_v, out_hbm.at[wid], sem).wait()


def _mlp_kernel(delta_ref, x_ref, w1t_ref, b1_ref, w2t_ref, b2_ref, out_ref):
    delta = delta_ref[0].reshape(KNN * RS, 4)
    h = jnp.maximum(
        jnp.dot(delta, w1t_ref[...], preferred_element_type=jnp.float32)
        + b1_ref[...], 0.0)
    pe = (jnp.dot(h, w2t_ref[...], preferred_element_type=jnp.float32)
          + b2_ref[...])
    out_ref[0] = x_ref[0] + pe.reshape(KNN, RS, D_M)


@jax.jit
def kernel(xyz, x, W1, b1, W2, b2):
    B = xyz.shape[0]
    btc = B - BSC
    pts = jnp.concatenate(
        [xyz, jnp.zeros((B, N, 5), dtype=xyz.dtype)], axis=-1)
    ptst = jnp.transpose(pts, (0, 2, 1))
    hi = pts.astype(jnp.bfloat16)
    lo = (pts - hi.astype(jnp.float32)).astype(jnp.bfloat16)
    hilo = jnp.concatenate([hi, lo], axis=-1)
    w1t8 = jnp.concatenate(
        [W1.T, jnp.zeros((5, D_M), dtype=W1.dtype)], axis=0)
    w1t4 = w1t8[:4]
    b1r = b1.reshape(1, D_M)
    b2r = b2.reshape(1, D_M)
    w2t = W2.T

    out_tc = pl.pallas_call(
        _tc_fused_kernel,
        grid=(btc, N // R),
        in_specs=[
            pl.BlockSpec((1, 8, N), lambda b_, i: (b_, 0, 0)),
            pl.BlockSpec((1, N, 16), lambda b_, i: (b_, 0, 0)),
            pl.BlockSpec((1, R, 8), lambda b_, i: (b_, i, 0)),
            pl.BlockSpec((1, KNN, R, D_M), lambda b_, i: (b_, 0, i, 0)),
            pl.BlockSpec((8, D_M), lambda b_, i: (0, 0)),
            pl.BlockSpec((1, D_M), lambda b_, i: (0, 0)),
            pl.BlockSpec((D_M, D_M), lambda b_, i: (0, 0)),
            pl.BlockSpec((1, D_M), lambda b_, i: (0, 0)),
        ],
        out_specs=pl.BlockSpec((1, KNN, R, D_M), lambda b_, i: (b_, 0, i, 0)),
        out_shape=jax.ShapeDtypeStruct((btc, KNN, N, D_M), x.dtype),
    )(ptst[:btc], hilo[:btc], pts[:btc], x[:btc], w1t8, b1r, w2t, b2r)

    dists = pl.pallas_call(
        _dist_kernel,
        grid=(BSC, N // R),
        in_specs=[
            pl.BlockSpec((1, 8, N), lambda b_, i: (b_, 0, 0)),
            pl.BlockSpec((1, R, 8), lambda b_, i: (b_, i, 0)),
        ],
        out_specs=pl.BlockSpec((1, R, N), lambda b_, i: (b_, i, 0)),
        out_shape=jax.ShapeDtypeStruct((BSC, N, N), jnp.float32),
    )(ptst[btc:], pts[btc:])

    mesh = plsc.VectorSubcoreMesh(core_axis_name="c", subcore_axis_name="s")
    sc_knn = functools.partial(
        pl.kernel,
        mesh=mesh,
        out_type=pltpu.HBM((NW, KNN * RS * 4), jnp.float32),
        scratch_types=[
            pltpu.VMEM((3, N), jnp.float32),
            pltpu.VMEM((N,), jnp.float32),
            pltpu.VMEM((N,), jnp.float32),
            pltpu.VMEM((N,), jnp.int32),
            pltpu.VMEM((KNN * RS * 4,), jnp.float32),
            pltpu.SemaphoreType.DMA,
            pltpu.SemaphoreType.DMA,
            pltpu.SemaphoreType.DMA,
        ],
        compiler_params=pltpu.CompilerParams(needs_layout_passes=False),
    )(_sc_knn_body)
    delta = sc_knn(dists, ptst[btc:, :3, :]).reshape(NW, KNN, RS, 4)

    nblk = N // RS
    out_sc = pl.pallas_call(
        _mlp_kernel,
        grid=(NW,),
        in_specs=[
            pl.BlockSpec((1, KNN, RS, 4), lambda w: (w, 0, 0, 0)),
            pl.BlockSpec((1, KNN, RS, D_M),
                         lambda w: (w // nblk, 0, w % nblk, 0)),
            pl.BlockSpec((4, D_M), lambda w: (0, 0)),
            pl.BlockSpec((1, D_M), lambda w: (0, 0)),
            pl.BlockSpec((D_M, D_M), lambda w: (0, 0)),
            pl.BlockSpec((1, D_M), lambda w: (0, 0)),
        ],
        out_specs=pl.BlockSpec((1, KNN, RS, D_M),
                               lambda w: (w // nblk, 0, w % nblk, 0)),
        out_shape=jax.ShapeDtypeStruct((BSC, KNN, N, D_M), x.dtype),
    )(delta, x[btc:], w1t4, b1r, w2t, b2r)

    return jnp.concatenate([out_tc, out_sc], axis=0)
